# Initial kernel scaffold; baseline (speedup 1.0000x reference)
import jax
import jax.numpy as jnp
from jax.experimental import pallas as pl


def kernel(z, x, edge_index):
    raise NotImplementedError("write your pallas kernel here")


# speedup vs baseline: 2.7598x; 2.7598x over previous
import jax
import jax.numpy as jnp
from jax import lax
from jax.experimental import pallas as pl
from jax.experimental.pallas import tpu as pltpu
from jax.experimental.pallas import tpu_sc as plsc

N_NODES = 10000
D_FEAT = 128
N_EDGES = 320000

NC = 2
NS = 16
L = 16
NW = NC * NS
EPW = N_EDGES // NW
B = 80
NCHUNK = EPW // B
GROUPS = B // L
K = D_FEAT // L

_DNUMS = lax.GatherDimensionNumbers(
    offset_dims=(), collapsed_slice_dims=(0,), start_index_map=(0,))


def _shuffle(v, perm):
    return lax.gather(v, perm[:, None], _DNUMS, (1,),
                      mode=lax.GatherScatterMode.PROMISE_IN_BOUNDS)


def _sc_body(z_hbm, x_hbm, src_hbm, dst_hbm, out_hbm,
             idx_s, idx_d, zrows, xrows, out_v, sem_z, sem_x):
    c = lax.axis_index("c")
    s = lax.axis_index("s")
    wid = s * NC + c
    base = wid * EPW

    pltpu.sync_copy(src_hbm.at[wid], idx_s)
    pltpu.sync_copy(dst_hbm.at[wid], idx_d)

    lanes = lax.iota(jnp.int32, L)
    perms = [lanes ^ (1 << k) for k in range(4)]
    masks = [(lanes & (1 << k)) == 0 for k in range(4)]

    def chunk_body(ci, carry):
        cz = pltpu.async_copy(z_hbm.at[idx_s.at[ci]], zrows, sem_z)
        cx = pltpu.async_copy(x_hbm.at[idx_d.at[ci]], xrows, sem_x)
        cz.wait()
        cx.wait()
        for g in range(GROUPS):
            vecs = []
            for e in range(L):
                row = g * L + e
                acc = zrows[row, pl.ds(0, L)] * xrows[row, pl.ds(0, L)]
                for k in range(1, K):
                    acc = acc + (zrows[row, pl.ds(k * L, L)]
                                 * xrows[row, pl.ds(k * L, L)])
                vecs.append(acc)
            for k in range(4):
                m, pm = masks[k], perms[k]
                vecs = [jnp.where(m, a, _shuffle(b, pm))
                        + jnp.where(m, _shuffle(a, pm), b)
                        for a, b in zip(vecs[0::2], vecs[1::2])]
            out_v[pl.ds(ci * B + g * L, L)] = vecs[0]
        return carry

    lax.fori_loop(0, NCHUNK, chunk_body, 0)
    pltpu.sync_copy(out_v, out_hbm.at[pl.ds(base, EPW)])


@jax.jit
def _decode(z, x, src, dst):
    mesh = plsc.VectorSubcoreMesh(core_axis_name="c", subcore_axis_name="s",
                                  num_cores=NC, num_subcores=NS)
    fn = pl.kernel(
        _sc_body,
        out_type=jax.ShapeDtypeStruct((N_EDGES,), jnp.float32),
        mesh=mesh,
        scratch_types=[
            pltpu.VMEM((NCHUNK, B), jnp.int32),
            pltpu.VMEM((NCHUNK, B), jnp.int32),
            pltpu.VMEM((B, D_FEAT), jnp.float32),
            pltpu.VMEM((B, D_FEAT), jnp.float32),
            pltpu.VMEM((EPW,), jnp.float32),
            pltpu.SemaphoreType.DMA,
            pltpu.SemaphoreType.DMA,
        ],
    )
    return fn(z, x, src, dst)


def kernel(z, x, edge_index):
    src = edge_index[0].reshape(NW, NCHUNK, B)
    dst = edge_index[1].reshape(NW, NCHUNK, B)
    return _decode(z, x, src, dst)


# speedup vs baseline: 4.0041x; 1.4509x over previous
import jax
import jax.numpy as jnp
from jax import lax
from jax.experimental import pallas as pl
from jax.experimental.pallas import tpu as pltpu
from jax.experimental.pallas import tpu_sc as plsc

N_NODES = 10000
D_FEAT = 128
N_EDGES = 320000

NC = 2
NS = 16
L = 16
NW = NC * NS
EPW = N_EDGES // NW
B = 80
NCHUNK = EPW // B
GROUPS = B // L
K = D_FEAT // L

_DNUMS = lax.GatherDimensionNumbers(
    offset_dims=(), collapsed_slice_dims=(0,), start_index_map=(0,))


def _shuffle(v, perm):
    return lax.gather(v, perm[:, None], _DNUMS, (1,),
                      mode=lax.GatherScatterMode.PROMISE_IN_BOUNDS)


def _sc_body(z_hbm, x_hbm, src_hbm, dst_hbm, out_hbm,
             idx_s, idx_d, zrows0, xrows0, zrows1, xrows1, out_v,
             sem_z0, sem_x0, sem_z1, sem_x1):
    c = lax.axis_index("c")
    s = lax.axis_index("s")
    wid = s * NC + c
    base = wid * EPW

    pltpu.sync_copy(src_hbm.at[wid], idx_s)
    pltpu.sync_copy(dst_hbm.at[wid], idx_d)

    lanes = lax.iota(jnp.int32, L)
    perms = [lanes ^ (1 << k) for k in range(4)]
    masks = [(lanes & (1 << k)) == 0 for k in range(4)]

    def issue(ci, zrows, xrows, sem_z, sem_x):
        pltpu.async_copy(z_hbm.at[idx_s.at[ci]], zrows, sem_z)
        pltpu.async_copy(x_hbm.at[idx_d.at[ci]], xrows, sem_x)

    def drain(zrows, xrows, sem_z, sem_x):
        pltpu.make_async_copy(z_hbm.at[idx_s.at[0]], zrows, sem_z).wait()
        pltpu.make_async_copy(x_hbm.at[idx_d.at[0]], xrows, sem_x).wait()

    def compute(ci, zrows, xrows):
        for g in range(GROUPS):
            vecs = []
            for e in range(L):
                row = g * L + e
                acc = zrows[row, pl.ds(0, L)] * xrows[row, pl.ds(0, L)]
                for k in range(1, K):
                    acc = acc + (zrows[row, pl.ds(k * L, L)]
                                 * xrows[row, pl.ds(k * L, L)])
                vecs.append(acc)
            for k in range(4):
                m, pm = masks[k], perms[k]
                vecs = [jnp.where(m, a, _shuffle(b, pm))
                        + jnp.where(m, _shuffle(a, pm), b)
                        for a, b in zip(vecs[0::2], vecs[1::2])]
            out_v[pl.ds(ci * B + g * L, L)] = vecs[0]

    issue(0, zrows0, xrows0, sem_z0, sem_x0)

    def chunk_pair(i, carry):
        ca = 2 * i
        issue(ca + 1, zrows1, xrows1, sem_z1, sem_x1)
        drain(zrows0, xrows0, sem_z0, sem_x0)
        compute(ca, zrows0, xrows0)
        issue(ca + 2, zrows0, xrows0, sem_z0, sem_x0)
        drain(zrows1, xrows1, sem_z1, sem_x1)
        compute(ca + 1, zrows1, xrows1)
        return carry

    lax.fori_loop(0, (NCHUNK - 1) // 2, chunk_pair, 0)
    drain(zrows0, xrows0, sem_z0, sem_x0)
    compute(NCHUNK - 1, zrows0, xrows0)

    pltpu.sync_copy(out_v, out_hbm.at[pl.ds(base, EPW)])


@jax.jit
def _decode(z, x, src, dst):
    mesh = plsc.VectorSubcoreMesh(core_axis_name="c", subcore_axis_name="s",
                                  num_cores=NC, num_subcores=NS)
    fn = pl.kernel(
        _sc_body,
        out_type=jax.ShapeDtypeStruct((N_EDGES,), jnp.float32),
        mesh=mesh,
        scratch_types=[
            pltpu.VMEM((NCHUNK, B), jnp.int32),
            pltpu.VMEM((NCHUNK, B), jnp.int32),
            pltpu.VMEM((B, D_FEAT), jnp.float32),
            pltpu.VMEM((B, D_FEAT), jnp.float32),
            pltpu.VMEM((B, D_FEAT), jnp.float32),
            pltpu.VMEM((B, D_FEAT), jnp.float32),
            pltpu.VMEM((EPW,), jnp.float32),
            pltpu.SemaphoreType.DMA,
            pltpu.SemaphoreType.DMA,
            pltpu.SemaphoreType.DMA,
            pltpu.SemaphoreType.DMA,
        ],
    )
    return fn(z, x, src, dst)


def kernel(z, x, edge_index):
    src = edge_index[0].reshape(NW, NCHUNK, B)
    dst = edge_index[1].reshape(NW, NCHUNK, B)
    return _decode(z, x, src, dst)


# speedup vs baseline: 7.2852x; 1.8194x over previous
import jax
import jax.numpy as jnp
from jax import lax
from jax.experimental import pallas as pl
from jax.experimental.pallas import tpu as pltpu
from jax.experimental.pallas import tpu_sc as plsc

N_NODES = 10000
D_FEAT = 128
N_EDGES = 320000

NC = 2
NS = 16
L = 16
NW = NC * NS
EPW = N_EDGES // NW
B = 80
NCHUNK = EPW // B
GROUPS = B // L
K = D_FEAT // L

_DNUMS = lax.GatherDimensionNumbers(
    offset_dims=(), collapsed_slice_dims=(0,), start_index_map=(0,))


def _shuffle(v, perm):
    return lax.gather(v, perm[:, None], _DNUMS, (1,),
                      mode=lax.GatherScatterMode.PROMISE_IN_BOUNDS)


def _sc_body(z_hbm, x_hbm, src_hbm, dst_hbm, out_hbm,
             idx_s, idx_d, zrows0, xrows0, zrows1, xrows1, zrows2, xrows2,
             out_v, sem_z0, sem_x0, sem_z1, sem_x1, sem_z2, sem_x2):
    c = lax.axis_index("c")
    s = lax.axis_index("s")
    wid = s * NC + c
    base = wid * EPW

    pltpu.sync_copy(src_hbm.at[wid], idx_s)
    pltpu.sync_copy(dst_hbm.at[wid], idx_d)

    lanes = lax.iota(jnp.int32, L)
    perms = [lanes ^ (1 << k) for k in range(4)]
    masks = [(lanes & (1 << k)) == 0 for k in range(4)]

    def issue(ci, zrows, xrows, sem_z, sem_x):
        pltpu.async_copy(z_hbm.at[idx_s.at[ci]], zrows, sem_z)
        pltpu.async_copy(x_hbm.at[idx_d.at[ci]], xrows, sem_x)

    def drain(zrows, xrows, sem_z, sem_x):
        pltpu.make_async_copy(z_hbm.at[idx_s.at[0]], zrows, sem_z).wait()
        pltpu.make_async_copy(x_hbm.at[idx_d.at[0]], xrows, sem_x).wait()

    def compute(ci, zrows, xrows):
        def g_body(g, carry):
            vecs = []
            for e in range(L):
                row = g * L + e
                acc = zrows[row, pl.ds(0, L)] * xrows[row, pl.ds(0, L)]
                for k in range(1, K):
                    acc = acc + (zrows[row, pl.ds(k * L, L)]
                                 * xrows[row, pl.ds(k * L, L)])
                vecs.append(acc)
            for k in range(4):
                m, pm = masks[k], perms[k]
                vecs = [jnp.where(m, a, _shuffle(b, pm))
                        + jnp.where(m, _shuffle(a, pm), b)
                        for a, b in zip(vecs[0::2], vecs[1::2])]
            out_v[pl.ds(ci * B + g * L, L)] = vecs[0]
            return carry

        lax.fori_loop(0, GROUPS, g_body, 0)

    bufs = [(zrows0, xrows0, sem_z0, sem_x0),
            (zrows1, xrows1, sem_z1, sem_x1),
            (zrows2, xrows2, sem_z2, sem_x2)]

    issue(0, *bufs[0])
    issue(1, *bufs[1])

    def chunk_tri(i, carry):
        ca = 3 * i
        issue(ca + 2, *bufs[2])
        drain(*bufs[0])
        compute(ca, bufs[0][0], bufs[0][1])
        issue(ca + 3, *bufs[0])
        drain(*bufs[1])
        compute(ca + 1, bufs[1][0], bufs[1][1])
        issue(ca + 4, *bufs[1])
        drain(*bufs[2])
        compute(ca + 2, bufs[2][0], bufs[2][1])
        return carry

    lax.fori_loop(0, (NCHUNK - 2) // 3, chunk_tri, 0)
    drain(*bufs[0])
    compute(NCHUNK - 2, bufs[0][0], bufs[0][1])
    drain(*bufs[1])
    compute(NCHUNK - 1, bufs[1][0], bufs[1][1])

    pltpu.sync_copy(out_v, out_hbm.at[pl.ds(base, EPW)])


@jax.jit
def _decode(z, x, src, dst):
    mesh = plsc.VectorSubcoreMesh(core_axis_name="c", subcore_axis_name="s",
                                  num_cores=NC, num_subcores=NS)
    fn = pl.kernel(
        _sc_body,
        out_type=jax.ShapeDtypeStruct((N_EDGES,), jnp.float32),
        mesh=mesh,
        scratch_types=[
            pltpu.VMEM((NCHUNK, B), jnp.int32),
            pltpu.VMEM((NCHUNK, B), jnp.int32),
            pltpu.VMEM((B, D_FEAT), jnp.float32),
            pltpu.VMEM((B, D_FEAT), jnp.float32),
            pltpu.VMEM((B, D_FEAT), jnp.float32),
            pltpu.VMEM((B, D_FEAT), jnp.float32),
            pltpu.VMEM((B, D_FEAT), jnp.float32),
            pltpu.VMEM((B, D_FEAT), jnp.float32),
            pltpu.VMEM((EPW,), jnp.float32),
            pltpu.SemaphoreType.DMA,
            pltpu.SemaphoreType.DMA,
            pltpu.SemaphoreType.DMA,
            pltpu.SemaphoreType.DMA,
            pltpu.SemaphoreType.DMA,
            pltpu.SemaphoreType.DMA,
        ],
    )
    return fn(z, x, src, dst)


def kernel(z, x, edge_index):
    src = edge_index[0].reshape(NW, NCHUNK, B)
    dst = edge_index[1].reshape(NW, NCHUNK, B)
    return _decode(z, x, src, dst)
